# baseline (speedup 1.0000x reference)
import functools

import jax
import jax.numpy as jnp
from jax.experimental import pallas as pl
from jax.experimental.pallas import tpu as pltpu
from jax.experimental.pallas import tpu_sc as plsc

_B, _N, _C, _H, _W = 8, 6, 128, 64, 176
_BN = _B * _N
_P = _H * _W
_HID = 64
_NC = 4
_K = 1000
_EPS = 1e-5
_TAPS = 9
_INV_COUNT = 1.0 / (_BN * _P)

_PREC = jax.lax.Precision.DEFAULT


def _conv_stats_body(x_ref, wd_ref, b1_ref, y_ref, s_ref, ss_ref, x3, q):
    i = pl.program_id(0)
    x = x_ref[0]
    wc = jax.lax.broadcasted_iota(jnp.int32, (1, _P), 1) % _W
    zcol = jnp.zeros((_C, 1), jnp.bfloat16)
    xm = jnp.concatenate([zcol, x[:, :_P - 1]], axis=1)
    xm = jnp.where(wc > 0, xm, jnp.bfloat16(0))
    xp = jnp.concatenate([x[:, 1:], zcol], axis=1)
    xp = jnp.where(wc < _W - 1, xp, jnp.bfloat16(0))
    x3[0:_C, :] = xm
    x3[_C:2 * _C, :] = x
    x3[2 * _C:3 * _C, :] = xp
    q[:, :] = jax.lax.dot_general(
        wd_ref[:, :], x3[:, :], (((1,), (0,)), ((), ())),
        precision=_PREC, preferred_element_type=jnp.float32)

    y_ref[0, :, :] = q[_HID:2 * _HID, :] + b1_ref[:, :]
    y_ref[0, :, _W:_P] += q[0:_HID, 0:_P - _W]
    y_ref[0, :, 0:_P - _W] += q[2 * _HID:3 * _HID, _W:_P]

    yv = y_ref[0]
    s = jnp.sum(yv, axis=1, keepdims=True)
    ss = jnp.sum(yv * yv, axis=1, keepdims=True)

    @pl.when(i == 0)
    def _():
        s_ref[:, :] = s
        ss_ref[:, :] = ss

    @pl.when(i > 0)
    def _():
        s_ref[:, :] += s
        ss_ref[:, :] += ss


def _head_body(y_ref, s_ref, ss_ref, g_ref, be_ref, w2_ref, b2_ref,
               probs_ref, sp_ref):
    mu = s_ref[:, :] * _INV_COUNT
    var = ss_ref[:, :] * _INV_COUNT - mu * mu
    scale = g_ref[:, :] / jnp.sqrt(var + _EPS)
    shift = be_ref[:, :] - mu * scale
    yn = jnp.maximum(y_ref[0] * scale + shift, 0.0)
    logits = jax.lax.dot_general(
        w2_ref[:, :], yn, (((1,), (0,)), ((), ())),
        precision=_PREC, preferred_element_type=jnp.float32) + b2_ref[:, :]
    mx = jnp.max(logits, axis=0, keepdims=True)
    e = jnp.exp(logits - mx)
    probs = e / jnp.sum(e, axis=0, keepdims=True)
    probs_ref[0] = probs

    scores = jnp.max(probs[1:_NC, :], axis=0, keepdims=True)
    nonempty = scores > probs[0:1, :]
    sp_ref[0] = jnp.where(nonempty, -scores, scores)


_NCHUNK = _P // 16
_UNROLL = 4


def _sc_select_image(img, sp_hbm, mask_hbm, sp_v, sabs_v, mask_v, tmp_v):
    pltpu.sync_copy(sp_hbm.at[img], sp_v)
    absmask = jnp.int32(0x7FFFFFFF)

    def p_abs(j, _):
        for u in range(_UNROLL):
            si = sp_v[pl.ds(j * 16 * _UNROLL + u * 16, 16)]
            sabs_v[pl.ds(j * 16 * _UNROLL + u * 16, 16)] = si & absmask
        return 0

    jax.lax.fori_loop(0, _NCHUNK // _UNROLL, p_abs, 0)

    kvec = jnp.full((16,), _K, jnp.int32)
    zvec = jnp.zeros((16,), jnp.int32)
    t = zvec
    for bit in range(29, -1, -1):
        cand = t | jnp.full((16,), 1 << bit, jnp.int32)

        def p_count(j, acc, cand=cand):
            for u in range(_UNROLL):
                s = sabs_v[pl.ds(j * 16 * _UNROLL + u * 16, 16)]
                acc = acc + jnp.where(s >= cand,
                                      jnp.full((16,), 1, jnp.int32),
                                      jnp.zeros((16,), jnp.int32))
            return acc

        acc = jax.lax.fori_loop(0, _NCHUNK // _UNROLL, p_count, zvec)
        for sh in (8, 4, 2, 1):
            tmp_v[pl.ds(0, 16)] = acc
            tmp_v[pl.ds(16, 16)] = acc
            acc = acc + tmp_v[pl.ds(sh, 16)]
        t = jnp.where(acc >= kvec, cand, t)

    fone = jnp.full((16,), 1.0, jnp.float32)
    fzero = jnp.zeros((16,), jnp.float32)

    def p_mask(j, _):
        for u in range(_UNROLL):
            si = sp_v[pl.ds(j * 16 * _UNROLL + u * 16, 16)]
            keep = ((si & absmask) >= t) & (si < jnp.int32(0))
            mask_v[pl.ds(j * 16 * _UNROLL + u * 16, 16)] = jnp.where(
                keep, fone, fzero)
        return 0

    jax.lax.fori_loop(0, _NCHUNK // _UNROLL, p_mask, 0)
    pltpu.sync_copy(mask_v, mask_hbm.at[img])


def _sc_select(sp_hbm, mask_hbm, sp_v, sabs_v, mask_v, tmp_v):
    c = jax.lax.axis_index("c")
    s = jax.lax.axis_index("s")
    wid = s * 2 + c
    _sc_select_image(wid, sp_hbm, mask_hbm, sp_v, sabs_v, mask_v, tmp_v)

    @pl.when(wid < _BN - 32)
    def _():
        _sc_select_image(wid + 32, sp_hbm, mask_hbm, sp_v, sabs_v, mask_v,
                         tmp_v)


@jax.jit
def kernel(image_features, W1, b1, gamma, beta, W2, b2):
    x = image_features.astype(jnp.bfloat16).reshape(_BN, _C, _P)
    wd = jnp.transpose(W1, (2, 0, 3, 1)).reshape(3 * _HID, 3 * _C)
    wd = wd.astype(jnp.bfloat16)
    b1c = b1.reshape(_HID, 1)
    gc = gamma.reshape(_HID, 1)
    bec = beta.reshape(_HID, 1)
    w2m = W2.reshape(_NC, _HID)
    b2c = b2.reshape(_NC, 1)

    f32 = jnp.float32
    y, s, ss = pl.pallas_call(
        _conv_stats_body,
        grid=(_BN,),
        in_specs=[
            pl.BlockSpec((1, _C, _P), lambda i: (i, 0, 0)),
            pl.BlockSpec((3 * _HID, 3 * _C), lambda i: (0, 0)),
            pl.BlockSpec((_HID, 1), lambda i: (0, 0)),
        ],
        out_specs=[
            pl.BlockSpec((1, _HID, _P), lambda i: (i, 0, 0)),
            pl.BlockSpec((_HID, 1), lambda i: (0, 0)),
            pl.BlockSpec((_HID, 1), lambda i: (0, 0)),
        ],
        out_shape=[
            jax.ShapeDtypeStruct((_BN, _HID, _P), f32),
            jax.ShapeDtypeStruct((_HID, 1), f32),
            jax.ShapeDtypeStruct((_HID, 1), f32),
        ],
        scratch_shapes=[pltpu.VMEM((3 * _C, _P), jnp.bfloat16),
                        pltpu.VMEM((3 * _HID, _P), f32)],
    )(x, wd, b1c)

    probs, spacked = pl.pallas_call(
        _head_body,
        grid=(_BN,),
        in_specs=[
            pl.BlockSpec((1, _HID, _P), lambda i: (i, 0, 0)),
            pl.BlockSpec((_HID, 1), lambda i: (0, 0)),
            pl.BlockSpec((_HID, 1), lambda i: (0, 0)),
            pl.BlockSpec((_HID, 1), lambda i: (0, 0)),
            pl.BlockSpec((_HID, 1), lambda i: (0, 0)),
            pl.BlockSpec((_NC, _HID), lambda i: (0, 0)),
            pl.BlockSpec((_NC, 1), lambda i: (0, 0)),
        ],
        out_specs=[
            pl.BlockSpec((1, _NC, _P), lambda i: (i, 0, 0)),
            pl.BlockSpec((1, 1, _P), lambda i: (i, 0, 0)),
        ],
        out_shape=[
            jax.ShapeDtypeStruct((_BN, _NC, _P), f32),
            jax.ShapeDtypeStruct((_BN, 1, _P), f32),
        ],
    )(y, s, ss, gc, bec, w2m, b2c)

    sc_select = functools.partial(
        pl.kernel,
        mesh=plsc.VectorSubcoreMesh(core_axis_name="c", subcore_axis_name="s"),
        out_type=jax.ShapeDtypeStruct((_BN, _P), f32),
        scratch_types=[
            pltpu.VMEM((_P,), jnp.int32),
            pltpu.VMEM((_P,), jnp.int32),
            pltpu.VMEM((_P,), f32),
            pltpu.VMEM((32,), jnp.int32),
        ],
    )(_sc_select)
    sp_bits = jax.lax.bitcast_convert_type(spacked.reshape(_BN, _P),
                                           jnp.int32)
    mask = sc_select(sp_bits)

    probs_out = probs.reshape(_B, _N, _NC, _H, _W)
    mask_out = mask.reshape(_B, _N, _H, _W).astype(jnp.bool_)
    return probs_out, mask_out
